# baseline (speedup 1.0000x reference)
import functools

import jax
import jax.numpy as jnp
from jax import lax
from jax.experimental import pallas as pl
from jax.experimental.pallas import tpu as pltpu
from jax.experimental.pallas import tpu_sc as plsc

_NC = 2
_NS = 16
_NW = _NC * _NS
_CH = 128


def _prep_body(x_ref, w1a_ref, w1b_ref, b1_ref, a_ref, b_ref):
    xb = x_ref[...]
    a_ref[...] = (
        jnp.dot(xb, w1a_ref[...], preferred_element_type=jnp.float32) + b1_ref[...]
    )
    b_ref[...] = jnp.dot(xb, w1b_ref[...], preferred_element_type=jnp.float32)


def _prep(x, w1a, w1b, b1):
    n, d = x.shape
    mid = w1a.shape[1]
    blk = 2000
    grid = n // blk
    return pl.pallas_call(
        _prep_body,
        grid=(grid,),
        in_specs=[
            pl.BlockSpec((blk, d), lambda i: (i, 0)),
            pl.BlockSpec((d, mid), lambda i: (0, 0)),
            pl.BlockSpec((d, mid), lambda i: (0, 0)),
            pl.BlockSpec((1, mid), lambda i: (0, 0)),
        ],
        out_specs=[
            pl.BlockSpec((blk, mid), lambda i: (i, 0)),
            pl.BlockSpec((blk, mid), lambda i: (i, 0)),
        ],
        out_shape=[
            jax.ShapeDtypeStruct((n, mid), jnp.float32),
            jax.ShapeDtypeStruct((n, mid), jnp.float32),
        ],
    )(x, w1a, w1b, b1)


_GCH = 200


def _gather(a, b, src, dst):
    mid = a.shape[1]
    e = src.shape[0]
    epw = e // _NW
    n_ch = epw // _GCH
    gh = _GCH // 2
    mesh = plsc.VectorSubcoreMesh(core_axis_name="c", subcore_axis_name="s")

    @functools.partial(
        pl.kernel,
        out_type=jax.ShapeDtypeStruct((e // 2, 2 * mid), jnp.float32),
        mesh=mesh,
        scratch_types=[
            pltpu.VMEM((epw,), jnp.int32),
            pltpu.VMEM((epw,), jnp.int32),
            pltpu.VMEM((2, _GCH, mid), jnp.float32),
            pltpu.VMEM((2, _GCH, mid), jnp.float32),
            pltpu.VMEM((2, gh, 2 * mid), jnp.float32),
            pltpu.SemaphoreType.DMA,
            pltpu.SemaphoreType.DMA,
            pltpu.SemaphoreType.DMA,
            pltpu.SemaphoreType.DMA,
        ],
        compiler_params=pltpu.CompilerParams(use_tc_tiling_on_sc=False),
    )
    def k(a_hbm, b_hbm, src_hbm, dst_hbm, g_hbm, idx_s, idx_d,
          buf_a, buf_b, buf_o, sg0, sg1, sw0, sw1):
        cid = lax.axis_index("c")
        sid = lax.axis_index("s")
        w = sid * _NC + cid
        base = w * epw
        sg = (sg0, sg1)
        sw = (sw0, sw1)

        pltpu.sync_copy(src_hbm.at[pl.ds(base, epw)], idx_s)
        pltpu.sync_copy(dst_hbm.at[pl.ds(base, epw)], idx_d)

        def start_gather(c, slot):
            off = c * _GCH
            pltpu.async_copy(
                a_hbm.at[idx_s.at[pl.ds(off, _GCH)]], buf_a.at[slot], sg[slot]
            )
            pltpu.async_copy(
                b_hbm.at[idx_d.at[pl.ds(off, _GCH)]], buf_b.at[slot], sg[slot]
            )

        def wait_gather(slot):
            pltpu.make_async_copy(
                a_hbm.at[pl.ds(0, _GCH)], buf_a.at[slot], sg[slot]
            ).wait()
            pltpu.make_async_copy(
                b_hbm.at[pl.ds(0, _GCH)], buf_b.at[slot], sg[slot]
            ).wait()

        def wait_write(slot):
            pltpu.make_async_copy(
                buf_o.at[slot], g_hbm.at[pl.ds(0, gh)], sw[slot]
            ).wait()

        def process(c, slot, first, do_issue):
            wait_gather(slot)
            if not first:
                wait_write(slot)

            def add_row(j2, _):
                for half in range(2):
                    for t in range(mid // 16):
                        buf_o[slot, j2, pl.ds(half * mid + t * 16, 16)] = (
                            buf_a[slot, 2 * j2 + half, pl.ds(t * 16, 16)]
                            + buf_b[slot, 2 * j2 + half, pl.ds(t * 16, 16)]
                        )
                return 0

            lax.fori_loop(0, gh, add_row, 0)
            if do_issue:
                start_gather(c + 2, slot)
            pltpu.async_copy(
                buf_o.at[slot],
                g_hbm.at[pl.ds(base // 2 + c * gh, gh)],
                sw[slot],
            )

        start_gather(0, 0)
        start_gather(1, 1)
        process(0, 0, True, True)
        process(1, 1, True, True)
        n_tail = 2 if n_ch % 2 == 0 else 3

        def body(i, _):
            c = 2 + i * 2
            process(c, 0, False, True)
            process(c + 1, 1, False, True)
            return 0

        lax.fori_loop(0, (n_ch - 2 - n_tail) // 2, body, 0)
        for c in range(n_ch - n_tail, n_ch):
            process(c, c % 2, False, c + 2 < n_ch)
        wait_write(0)
        wait_write(1)

    return k(a, b, src, dst)


def _edge_body(mid, g_ref, w2_ref, b2_ref, lng_ref, lnb_ref,
               he_ref, ho_ref, ssum_ref, ssq_ref):
    g = g_ref[...]
    u = jnp.where(g > 0, g, jnp.exp(jnp.minimum(g, 0.0)) - 1.0)

    @pl.when(pl.program_id(0) == 0)
    def _():
        ssum_ref[...] = jnp.zeros_like(ssum_ref)
        ssq_ref[...] = jnp.zeros_like(ssq_ref)

    for half, out_ref in ((0, he_ref), (1, ho_ref)):
        v = u[:, half * mid:(half + 1) * mid]
        mu = jnp.mean(v, axis=-1, keepdims=True)
        dev = v - mu
        var = jnp.mean(dev * dev, axis=-1, keepdims=True)
        ln = dev * lax.rsqrt(var + 1e-5) * lng_ref[...] + lnb_ref[...]
        h = (
            jnp.dot(ln, w2_ref[...], preferred_element_type=jnp.float32)
            + b2_ref[...]
        )
        h = jnp.where(h > 0, h, jnp.exp(jnp.minimum(h, 0.0)) - 1.0)
        out_ref[...] = h
        ssum_ref[...] += jnp.sum(h, axis=0, keepdims=True)
        ssq_ref[...] += jnp.sum(h * h, axis=0, keepdims=True)


def _edge(g2, w2, b2, lng, lnb):
    e2, mid2 = g2.shape
    mid = mid2 // 2
    d = w2.shape[1]
    blk = 2000
    grid = e2 // blk
    return pl.pallas_call(
        functools.partial(_edge_body, mid),
        grid=(grid,),
        in_specs=[
            pl.BlockSpec((blk, mid2), lambda i: (i, 0)),
            pl.BlockSpec((mid, d), lambda i: (0, 0)),
            pl.BlockSpec((1, d), lambda i: (0, 0)),
            pl.BlockSpec((1, mid), lambda i: (0, 0)),
            pl.BlockSpec((1, mid), lambda i: (0, 0)),
        ],
        out_specs=[
            pl.BlockSpec((blk, d), lambda i: (i, 0)),
            pl.BlockSpec((blk, d), lambda i: (i, 0)),
            pl.BlockSpec((1, d), lambda i: (0, 0)),
            pl.BlockSpec((1, d), lambda i: (0, 0)),
        ],
        out_shape=[
            jax.ShapeDtypeStruct((e2, d), jnp.float32),
            jax.ShapeDtypeStruct((e2, d), jnp.float32),
            jax.ShapeDtypeStruct((1, d), jnp.float32),
            jax.ShapeDtypeStruct((1, d), jnp.float32),
        ],
        compiler_params=pltpu.CompilerParams(
            dimension_semantics=("arbitrary",),
        ),
    )(g2, w2, b2, lng, lnb)


def _scatter(parts, n):
    d = parts[0][0].shape[1]
    rows_per_tile = n // _NS
    chunks = []
    left = rows_per_tile
    while left > 0:
        chunks.append(min(_CH, left))
        left -= chunks[-1]
    mesh = plsc.VectorSubcoreMesh(core_axis_name="c", subcore_axis_name="s")
    part_rows = [p[1].shape[0] for p in parts]

    @functools.partial(
        pl.kernel,
        out_type=[
            jax.ShapeDtypeStruct((_NC * n, d), jnp.float32),
            jax.ShapeDtypeStruct((_NC * n, 16), jnp.float32),
        ],
        mesh=mesh,
        scratch_types=[
            pltpu.VMEM((2, _CH), jnp.int32),
            pltpu.VMEM((2, _CH, d), jnp.float32),
            pltpu.VMEM((_CH, 16), jnp.float32),
            pltpu.VMEM_SHARED((n, d), jnp.float32),
            pltpu.VMEM_SHARED((n, 16), jnp.float32),
            pltpu.SemaphoreType.DMA,
            pltpu.SemaphoreType.DMA,
            pltpu.SemaphoreType.DMA,
            pltpu.SemaphoreType.DMA,
        ],
        compiler_params=pltpu.CompilerParams(use_tc_tiling_on_sc=False),
    )
    def k(*args):
        hbm = args[:2 * len(parts)]
        (s_out, deg_out, idx_v, rows_v, ones_v,
         s_sh, deg_sh, sl0, sl1, ss0, ss1) = args[2 * len(parts):]
        cid = lax.axis_index("c")
        sid = lax.axis_index("s")
        w = sid * _NC + cid
        lsem = (sl0, sl1)
        ssem = (ss0, ss1)

        def zero_o(i, _):
            ones_v[i, :] = jnp.zeros((16,), jnp.float32)
            return 0

        lax.fori_loop(0, _CH, zero_o, 0)

        def zero_z(i, _):
            for t in range(d // 16):
                rows_v[0, i, pl.ds(t * 16, 16)] = jnp.zeros((16,), jnp.float32)
            return 0

        lax.fori_loop(0, _CH, zero_z, 0)

        r0 = sid * rows_per_tile
        off = 0
        for cr in chunks:
            pltpu.sync_copy(rows_v.at[0, pl.ds(0, cr)], s_sh.at[pl.ds(r0 + off, cr)])
            pltpu.sync_copy(ones_v.at[pl.ds(0, cr)], deg_sh.at[pl.ds(r0 + off, cr)])
            off += cr

        def fill_ones(i, _):
            ones_v[i, :] = jnp.ones((16,), jnp.float32)
            return 0

        lax.fori_loop(0, _CH, fill_ones, 0)
        plsc.subcore_barrier()

        def run_part(h2_hbm, dst_hbm, nrow):
            nbase, extra = nrow // _NW, nrow % _NW
            n_pipe = nbase - (nbase % 2)

            def start_load(i, b):
                r = w + i * _NW
                pltpu.async_copy(dst_hbm.at[r], idx_v.at[b], lsem[b])
                pltpu.async_copy(
                    h2_hbm.at[pl.ds(r * _CH, _CH)], rows_v.at[b], lsem[b]
                )

            def process(i, b, do_issue):
                pltpu.make_async_copy(dst_hbm.at[0], idx_v.at[b], lsem[b]).wait()
                pltpu.make_async_copy(
                    h2_hbm.at[pl.ds(0, _CH)], rows_v.at[b], lsem[b]
                ).wait()
                pltpu.async_copy(
                    rows_v.at[b], s_sh.at[idx_v.at[b]], ssem[b], add=True
                )
                pltpu.async_copy(ones_v, deg_sh.at[idx_v.at[b]], ssem[b], add=True)
                pltpu.make_async_copy(
                    rows_v.at[b], s_sh.at[pl.ds(0, _CH)], ssem[b]
                ).wait()
                pltpu.make_async_copy(
                    ones_v, deg_sh.at[pl.ds(0, _CH)], ssem[b]
                ).wait()
                if do_issue:
                    start_load(i + 2, b)

            def sync_chunk(r):
                pltpu.sync_copy(dst_hbm.at[r], idx_v.at[0])
                pltpu.sync_copy(h2_hbm.at[pl.ds(r * _CH, _CH)], rows_v.at[0])
                pltpu.sync_copy(rows_v.at[0], s_sh.at[idx_v.at[0]], add=True)
                pltpu.sync_copy(ones_v, deg_sh.at[idx_v.at[0]], add=True)

            if n_pipe >= 4:
                start_load(0, 0)
                start_load(1, 1)
                process(0, 0, True)
                process(1, 1, True)

                def body(j, _):
                    i = 2 + j * 2
                    process(i, 0, True)
                    process(i + 1, 1, True)
                    return 0

                lax.fori_loop(0, (n_pipe - 4) // 2, body, 0)
                process(n_pipe - 2, 0, False)
                process(n_pipe - 1, 1, False)
            else:

                def sbody(i, _):
                    sync_chunk(w + i * _NW)
                    return 0

                lax.fori_loop(0, n_pipe, sbody, 0)

            for i in range(n_pipe, nbase):
                sync_chunk(w + i * _NW)

            @pl.when(w < extra)
            def _():
                sync_chunk(nbase * _NW + w)

        for pi in range(len(parts)):
            run_part(hbm[2 * pi], hbm[2 * pi + 1], part_rows[pi])

        plsc.subcore_barrier()

        off = 0
        for cr in chunks:
            pltpu.sync_copy(s_sh.at[pl.ds(r0 + off, cr)], rows_v.at[0, pl.ds(0, cr)])
            pltpu.sync_copy(
                rows_v.at[0, pl.ds(0, cr)],
                s_out.at[pl.ds(cid * n + r0 + off, cr)],
            )
            pltpu.sync_copy(deg_sh.at[pl.ds(r0 + off, cr)], ones_v.at[pl.ds(0, cr)])
            pltpu.sync_copy(
                ones_v.at[pl.ds(0, cr)],
                deg_out.at[pl.ds(cid * n + r0 + off, cr)],
            )
            off += cr

    return k(*[a for p in parts for a in p])


def _final_body(inv_e, nsp, nh, *refs):
    s_refs = refs[:nsp]
    d_refs = refs[nsp:2 * nsp]
    ss_refs = refs[2 * nsp:2 * nsp + nh]
    sq_refs = refs[2 * nsp + nh:2 * nsp + 2 * nh]
    bng_ref, bnb_ref, wr_ref, br_ref, x_ref, out_ref = refs[2 * nsp + 2 * nh:]
    s = s_refs[0][...]
    for r in s_refs[1:]:
        s = s + r[...]
    deg = d_refs[0][...][:, :1]
    for r in d_refs[1:]:
        deg = deg + r[...][:, :1]
    ssum = ss_refs[0][...]
    for r in ss_refs[1:]:
        ssum = ssum + r[...]
    ssq = sq_refs[0][...]
    for r in sq_refs[1:]:
        ssq = ssq + r[...]
    mean = ssum * inv_e
    var = ssq * inv_e - mean * mean
    a = bng_ref[...] * lax.rsqrt(var + 1e-5)
    c = bnb_ref[...] - mean * a
    rows = s * a + deg * c
    out_ref[...] = (
        jnp.dot(rows, wr_ref[...], preferred_element_type=jnp.float32)
        + br_ref[...]
        + x_ref[...]
    )


def _final(sps, degps, ssums, ssqs, bng, bnb, wr, br, x, e):
    n, d = x.shape
    blk = 2000
    grid = n // blk
    nblk = n // blk
    s_specs, d_specs = [], []
    for _ in sps:
        s_specs.append(pl.BlockSpec((blk, d), lambda i: (i, 0)))
        s_specs.append(pl.BlockSpec((blk, d), lambda i, _n=nblk: (i + _n, 0)))
    for _ in degps:
        d_specs.append(pl.BlockSpec((blk, 16), lambda i: (i, 0)))
        d_specs.append(pl.BlockSpec((blk, 16), lambda i, _n=nblk: (i + _n, 0)))
    st_specs = [
        pl.BlockSpec((1, d), lambda i: (0, 0))
        for _ in range(len(ssums) + len(ssqs))
    ]
    s_args = [a for p in sps for a in (p, p)]
    d_args = [a for p in degps for a in (p, p)]
    return pl.pallas_call(
        functools.partial(
            _final_body, float(1.0 / e), 2 * len(sps), len(ssums)
        ),
        grid=(grid,),
        in_specs=s_specs + d_specs + st_specs + [
            pl.BlockSpec((1, d), lambda i: (0, 0)),
            pl.BlockSpec((1, d), lambda i: (0, 0)),
            pl.BlockSpec((d, d), lambda i: (0, 0)),
            pl.BlockSpec((1, d), lambda i: (0, 0)),
            pl.BlockSpec((blk, d), lambda i: (i, 0)),
        ],
        out_specs=pl.BlockSpec((blk, d), lambda i: (i, 0)),
        out_shape=jax.ShapeDtypeStruct((n, d), jnp.float32),
    )(*s_args, *d_args, *ssums, *ssqs, bng, bnb, wr, br, x)


def kernel(x, W1, b1, ln_g, ln_b, W2, b2, bn_g, bn_b, Wr, br, edge_index):
    n, d = x.shape
    mid = W1.shape[1]
    e = edge_index.shape[1]
    nh = 2
    eh = e // nh
    a_nodes, b_nodes = _prep(x, W1[:d], W1[d:], b1.reshape(1, mid))
    sps, degps, ssums, ssqs = [], [], [], []
    for i in range(nh):
        src = lax.slice_in_dim(edge_index[0], i * eh, (i + 1) * eh)
        dst = lax.slice_in_dim(edge_index[1], i * eh, (i + 1) * eh)
        g2 = _gather(a_nodes, b_nodes, src, dst)
        he, ho, ssum, ssq = _edge(
            g2, W2, b2.reshape(1, d), ln_g.reshape(1, mid), ln_b.reshape(1, mid)
        )
        dpair = dst.reshape(eh // 2, 2).T
        sp, degp = _scatter(
            [
                (he, dpair[0].reshape(eh // 2 // _CH, _CH)),
                (ho, dpair[1].reshape(eh // 2 // _CH, _CH)),
            ],
            n,
        )
        sps.append(sp)
        degps.append(degp)
        ssums.append(ssum)
        ssqs.append(ssq)
    return _final(
        sps, degps, ssums, ssqs, bn_g.reshape(1, d), bn_b.reshape(1, d),
        Wr, br.reshape(1, d), x, e,
    )


# speedup vs baseline: 1.6121x; 1.6121x over previous
import functools

import jax
import jax.numpy as jnp
from jax import lax
from jax.experimental import pallas as pl
from jax.experimental.pallas import tpu as pltpu
from jax.experimental.pallas import tpu_sc as plsc

_NC = 2
_NS = 16
_NW = _NC * _NS
_CH = 128


def _prep_body(x_ref, w1a_ref, w1b_ref, b1_ref, a_ref, b_ref):
    xb = x_ref[...]
    a_ref[...] = (
        jnp.dot(xb, w1a_ref[...], preferred_element_type=jnp.float32) + b1_ref[...]
    )
    b_ref[...] = jnp.dot(xb, w1b_ref[...], preferred_element_type=jnp.float32)


def _prep(x, w1a, w1b, b1):
    n, d = x.shape
    mid = w1a.shape[1]
    blk = 2000
    grid = n // blk
    return pl.pallas_call(
        _prep_body,
        grid=(grid,),
        in_specs=[
            pl.BlockSpec((blk, d), lambda i: (i, 0)),
            pl.BlockSpec((d, mid), lambda i: (0, 0)),
            pl.BlockSpec((d, mid), lambda i: (0, 0)),
            pl.BlockSpec((1, mid), lambda i: (0, 0)),
        ],
        out_specs=[
            pl.BlockSpec((blk, mid), lambda i: (i, 0)),
            pl.BlockSpec((blk, mid), lambda i: (i, 0)),
        ],
        out_shape=[
            jax.ShapeDtypeStruct((n, mid), jnp.float32),
            jax.ShapeDtypeStruct((n, mid), jnp.float32),
        ],
    )(x, w1a, w1b, b1)


_GCH = 200


def _gather(a, b, src, dst):
    mid = a.shape[1]
    e = src.shape[0]
    epw = e // _NW
    n_ch = epw // _GCH
    mesh = plsc.VectorSubcoreMesh(core_axis_name="c", subcore_axis_name="s")

    @functools.partial(
        pl.kernel,
        out_type=jax.ShapeDtypeStruct((e, 128), jnp.float32),
        mesh=mesh,
        scratch_types=[
            pltpu.VMEM((epw,), jnp.int32),
            pltpu.VMEM((epw,), jnp.int32),
            pltpu.VMEM((2, _GCH, mid), jnp.float32),
            pltpu.VMEM((2, _GCH, mid), jnp.float32),
            pltpu.VMEM((2, _GCH, 128), jnp.float32),
            pltpu.SemaphoreType.DMA,
            pltpu.SemaphoreType.DMA,
            pltpu.SemaphoreType.DMA,
            pltpu.SemaphoreType.DMA,
        ],
        compiler_params=pltpu.CompilerParams(use_tc_tiling_on_sc=False),
    )
    def k(a_hbm, b_hbm, src_hbm, dst_hbm, g_hbm, idx_s, idx_d,
          buf_a, buf_b, buf_o, sg0, sg1, sw0, sw1):
        cid = lax.axis_index("c")
        sid = lax.axis_index("s")
        w = sid * _NC + cid
        base = w * epw
        sg = (sg0, sg1)
        sw = (sw0, sw1)

        pltpu.sync_copy(src_hbm.at[pl.ds(base, epw)], idx_s)
        pltpu.sync_copy(dst_hbm.at[pl.ds(base, epw)], idx_d)

        def start_gather(c, slot):
            off = c * _GCH
            pltpu.async_copy(
                a_hbm.at[idx_s.at[pl.ds(off, _GCH)]], buf_a.at[slot], sg[slot]
            )
            pltpu.async_copy(
                b_hbm.at[idx_d.at[pl.ds(off, _GCH)]], buf_b.at[slot], sg[slot]
            )

        def wait_gather(slot):
            pltpu.make_async_copy(
                a_hbm.at[pl.ds(0, _GCH)], buf_a.at[slot], sg[slot]
            ).wait()
            pltpu.make_async_copy(
                b_hbm.at[pl.ds(0, _GCH)], buf_b.at[slot], sg[slot]
            ).wait()

        def wait_write(slot):
            pltpu.make_async_copy(
                buf_o.at[slot], g_hbm.at[pl.ds(0, _GCH)], sw[slot]
            ).wait()

        def process(c, slot, first, do_issue):
            wait_gather(slot)
            if not first:
                wait_write(slot)

            def add_row(j, _):
                for t in range(mid // 16):
                    buf_o[slot, j, pl.ds(t * 16, 16)] = (
                        buf_a[slot, j, pl.ds(t * 16, 16)]
                        + buf_b[slot, j, pl.ds(t * 16, 16)]
                    )
                return 0

            lax.fori_loop(0, _GCH, add_row, 0)
            if do_issue:
                start_gather(c + 2, slot)
            pltpu.async_copy(
                buf_o.at[slot],
                g_hbm.at[pl.ds(base + c * _GCH, _GCH)],
                sw[slot],
            )

        start_gather(0, 0)
        start_gather(1, 1)
        process(0, 0, True, True)
        process(1, 1, True, True)
        n_tail = 2 if n_ch % 2 == 0 else 3

        def body(i, _):
            c = 2 + i * 2
            process(c, 0, False, True)
            process(c + 1, 1, False, True)
            return 0

        lax.fori_loop(0, (n_ch - 2 - n_tail) // 2, body, 0)
        for c in range(n_ch - n_tail, n_ch):
            process(c, c % 2, False, c + 2 < n_ch)
        wait_write(0)
        wait_write(1)

    return k(a, b, src, dst)


def _edge_body(mid, g_ref, w2_ref, b2_ref, lng_ref, lnb_ref,
               h2_ref, ssum_ref, ssq_ref):
    g = g_ref[...][:, :mid]
    u = jnp.where(g > 0, g, jnp.exp(jnp.minimum(g, 0.0)) - 1.0)
    mu = jnp.mean(u, axis=-1, keepdims=True)
    dev = u - mu
    var = jnp.mean(dev * dev, axis=-1, keepdims=True)
    ln = dev * lax.rsqrt(var + 1e-5) * lng_ref[...] + lnb_ref[...]
    h = jnp.dot(ln, w2_ref[...], preferred_element_type=jnp.float32) + b2_ref[...]
    h = jnp.where(h > 0, h, jnp.exp(jnp.minimum(h, 0.0)) - 1.0)
    h2_ref[...] = h

    @pl.when(pl.program_id(0) == 0)
    def _():
        ssum_ref[...] = jnp.zeros_like(ssum_ref)
        ssq_ref[...] = jnp.zeros_like(ssq_ref)

    ssum_ref[...] += jnp.sum(h, axis=0, keepdims=True)
    ssq_ref[...] += jnp.sum(h * h, axis=0, keepdims=True)


def _edge(g2, w2, b2, lng, lnb):
    e2, mid2 = g2.shape
    mid = mid2 // 2
    d = w2.shape[1]
    blk = 2000
    grid = e2 // blk
    return pl.pallas_call(
        functools.partial(_edge_body, mid),
        grid=(grid,),
        in_specs=[
            pl.BlockSpec((blk, mid2), lambda i: (i, 0)),
            pl.BlockSpec((mid, d), lambda i: (0, 0)),
            pl.BlockSpec((1, d), lambda i: (0, 0)),
            pl.BlockSpec((1, mid), lambda i: (0, 0)),
            pl.BlockSpec((1, mid), lambda i: (0, 0)),
        ],
        out_specs=[
            pl.BlockSpec((blk, d), lambda i: (i, 0)),
            pl.BlockSpec((1, d), lambda i: (0, 0)),
            pl.BlockSpec((1, d), lambda i: (0, 0)),
        ],
        out_shape=[
            jax.ShapeDtypeStruct((e2, d), jnp.float32),
            jax.ShapeDtypeStruct((1, d), jnp.float32),
            jax.ShapeDtypeStruct((1, d), jnp.float32),
        ],
        compiler_params=pltpu.CompilerParams(
            dimension_semantics=("arbitrary",),
        ),
    )(g2, w2, b2, lng, lnb)


def _scatter(parts, n):
    d = parts[0][0].shape[1]
    rows_per_tile = n // _NS
    chunks = []
    left = rows_per_tile
    while left > 0:
        chunks.append(min(_CH, left))
        left -= chunks[-1]
    mesh = plsc.VectorSubcoreMesh(core_axis_name="c", subcore_axis_name="s")
    part_rows = [p[1].shape[0] for p in parts]

    @functools.partial(
        pl.kernel,
        out_type=[
            jax.ShapeDtypeStruct((_NC * n, d), jnp.float32),
            jax.ShapeDtypeStruct((_NC * n, 16), jnp.float32),
        ],
        mesh=mesh,
        scratch_types=[
            pltpu.VMEM((2, _CH), jnp.int32),
            pltpu.VMEM((2, _CH, d), jnp.float32),
            pltpu.VMEM((_CH, 16), jnp.float32),
            pltpu.VMEM_SHARED((n, d), jnp.float32),
            pltpu.VMEM_SHARED((n, 16), jnp.float32),
            pltpu.SemaphoreType.DMA,
            pltpu.SemaphoreType.DMA,
            pltpu.SemaphoreType.DMA,
            pltpu.SemaphoreType.DMA,
        ],
        compiler_params=pltpu.CompilerParams(use_tc_tiling_on_sc=False),
    )
    def k(*args):
        hbm = args[:2 * len(parts)]
        (s_out, deg_out, idx_v, rows_v, ones_v,
         s_sh, deg_sh, sl0, sl1, ss0, ss1) = args[2 * len(parts):]
        cid = lax.axis_index("c")
        sid = lax.axis_index("s")
        w = sid * _NC + cid
        lsem = (sl0, sl1)
        ssem = (ss0, ss1)

        def zero_o(i, _):
            ones_v[i, :] = jnp.zeros((16,), jnp.float32)
            return 0

        lax.fori_loop(0, _CH, zero_o, 0)

        def zero_z(i, _):
            for t in range(d // 16):
                rows_v[0, i, pl.ds(t * 16, 16)] = jnp.zeros((16,), jnp.float32)
            return 0

        lax.fori_loop(0, _CH, zero_z, 0)

        r0 = sid * rows_per_tile
        off = 0
        for cr in chunks:
            pltpu.sync_copy(rows_v.at[0, pl.ds(0, cr)], s_sh.at[pl.ds(r0 + off, cr)])
            pltpu.sync_copy(ones_v.at[pl.ds(0, cr)], deg_sh.at[pl.ds(r0 + off, cr)])
            off += cr

        def fill_ones(i, _):
            ones_v[i, :] = jnp.ones((16,), jnp.float32)
            return 0

        lax.fori_loop(0, _CH, fill_ones, 0)
        plsc.subcore_barrier()

        def run_part(h2_hbm, dst_hbm, nrow):
            nbase, extra = nrow // _NW, nrow % _NW
            n_pipe = nbase - (nbase % 2)

            def start_load(i, b):
                r = w + i * _NW
                pltpu.async_copy(dst_hbm.at[r], idx_v.at[b], lsem[b])
                pltpu.async_copy(
                    h2_hbm.at[pl.ds(r * _CH, _CH)], rows_v.at[b], lsem[b]
                )

            def process(i, b, do_issue):
                pltpu.make_async_copy(dst_hbm.at[0], idx_v.at[b], lsem[b]).wait()
                pltpu.make_async_copy(
                    h2_hbm.at[pl.ds(0, _CH)], rows_v.at[b], lsem[b]
                ).wait()
                pltpu.async_copy(
                    rows_v.at[b], s_sh.at[idx_v.at[b]], ssem[b], add=True
                )
                pltpu.async_copy(ones_v, deg_sh.at[idx_v.at[b]], ssem[b], add=True)
                pltpu.make_async_copy(
                    rows_v.at[b], s_sh.at[pl.ds(0, _CH)], ssem[b]
                ).wait()
                pltpu.make_async_copy(
                    ones_v, deg_sh.at[pl.ds(0, _CH)], ssem[b]
                ).wait()
                if do_issue:
                    start_load(i + 2, b)

            def sync_chunk(r):
                pltpu.sync_copy(dst_hbm.at[r], idx_v.at[0])
                pltpu.sync_copy(h2_hbm.at[pl.ds(r * _CH, _CH)], rows_v.at[0])
                pltpu.sync_copy(rows_v.at[0], s_sh.at[idx_v.at[0]], add=True)
                pltpu.sync_copy(ones_v, deg_sh.at[idx_v.at[0]], add=True)

            if n_pipe >= 4:
                start_load(0, 0)
                start_load(1, 1)
                process(0, 0, True)
                process(1, 1, True)

                def body(j, _):
                    i = 2 + j * 2
                    process(i, 0, True)
                    process(i + 1, 1, True)
                    return 0

                lax.fori_loop(0, (n_pipe - 4) // 2, body, 0)
                process(n_pipe - 2, 0, False)
                process(n_pipe - 1, 1, False)
            else:

                def sbody(i, _):
                    sync_chunk(w + i * _NW)
                    return 0

                lax.fori_loop(0, n_pipe, sbody, 0)

            for i in range(n_pipe, nbase):
                sync_chunk(w + i * _NW)

            @pl.when(w < extra)
            def _():
                sync_chunk(nbase * _NW + w)

        for pi in range(len(parts)):
            run_part(hbm[2 * pi], hbm[2 * pi + 1], part_rows[pi])

        plsc.subcore_barrier()

        off = 0
        for cr in chunks:
            pltpu.sync_copy(s_sh.at[pl.ds(r0 + off, cr)], rows_v.at[0, pl.ds(0, cr)])
            pltpu.sync_copy(
                rows_v.at[0, pl.ds(0, cr)],
                s_out.at[pl.ds(cid * n + r0 + off, cr)],
            )
            pltpu.sync_copy(deg_sh.at[pl.ds(r0 + off, cr)], ones_v.at[pl.ds(0, cr)])
            pltpu.sync_copy(
                ones_v.at[pl.ds(0, cr)],
                deg_out.at[pl.ds(cid * n + r0 + off, cr)],
            )
            off += cr

    return k(*[a for p in parts for a in p])


def _final_body(inv_e, nsp, nh, *refs):
    s_refs = refs[:nsp]
    d_refs = refs[nsp:2 * nsp]
    ss_refs = refs[2 * nsp:2 * nsp + nh]
    sq_refs = refs[2 * nsp + nh:2 * nsp + 2 * nh]
    bng_ref, bnb_ref, wr_ref, br_ref, x_ref, out_ref = refs[2 * nsp + 2 * nh:]
    s = s_refs[0][...]
    for r in s_refs[1:]:
        s = s + r[...]
    deg = d_refs[0][...][:, :1]
    for r in d_refs[1:]:
        deg = deg + r[...][:, :1]
    ssum = ss_refs[0][...]
    for r in ss_refs[1:]:
        ssum = ssum + r[...]
    ssq = sq_refs[0][...]
    for r in sq_refs[1:]:
        ssq = ssq + r[...]
    mean = ssum * inv_e
    var = ssq * inv_e - mean * mean
    a = bng_ref[...] * lax.rsqrt(var + 1e-5)
    c = bnb_ref[...] - mean * a
    rows = s * a + deg * c
    out_ref[...] = (
        jnp.dot(rows, wr_ref[...], preferred_element_type=jnp.float32)
        + br_ref[...]
        + x_ref[...]
    )


def _final(sps, degps, ssums, ssqs, bng, bnb, wr, br, x, e):
    n, d = x.shape
    blk = 2000
    grid = n // blk
    nblk = n // blk
    s_specs, d_specs = [], []
    for _ in sps:
        s_specs.append(pl.BlockSpec((blk, d), lambda i: (i, 0)))
        s_specs.append(pl.BlockSpec((blk, d), lambda i, _n=nblk: (i + _n, 0)))
    for _ in degps:
        d_specs.append(pl.BlockSpec((blk, 16), lambda i: (i, 0)))
        d_specs.append(pl.BlockSpec((blk, 16), lambda i, _n=nblk: (i + _n, 0)))
    st_specs = [
        pl.BlockSpec((1, d), lambda i: (0, 0))
        for _ in range(len(ssums) + len(ssqs))
    ]
    s_args = [a for p in sps for a in (p, p)]
    d_args = [a for p in degps for a in (p, p)]
    return pl.pallas_call(
        functools.partial(
            _final_body, float(1.0 / e), 2 * len(sps), len(ssums)
        ),
        grid=(grid,),
        in_specs=s_specs + d_specs + st_specs + [
            pl.BlockSpec((1, d), lambda i: (0, 0)),
            pl.BlockSpec((1, d), lambda i: (0, 0)),
            pl.BlockSpec((d, d), lambda i: (0, 0)),
            pl.BlockSpec((1, d), lambda i: (0, 0)),
            pl.BlockSpec((blk, d), lambda i: (i, 0)),
        ],
        out_specs=pl.BlockSpec((blk, d), lambda i: (i, 0)),
        out_shape=jax.ShapeDtypeStruct((n, d), jnp.float32),
    )(*s_args, *d_args, *ssums, *ssqs, bng, bnb, wr, br, x)


def kernel(x, W1, b1, ln_g, ln_b, W2, b2, bn_g, bn_b, Wr, br, edge_index):
    n, d = x.shape
    mid = W1.shape[1]
    e = edge_index.shape[1]
    nh = 2
    eh = e // nh
    a_nodes, b_nodes = _prep(x, W1[:d], W1[d:], b1.reshape(1, mid))
    sps, degps, ssums, ssqs = [], [], [], []
    for i in range(nh):
        src = lax.slice_in_dim(edge_index[0], i * eh, (i + 1) * eh)
        dst = lax.slice_in_dim(edge_index[1], i * eh, (i + 1) * eh)
        g2 = _gather(a_nodes, b_nodes, src, dst)
        h2, ssum, ssq = _edge(
            g2, W2, b2.reshape(1, d), ln_g.reshape(1, mid), ln_b.reshape(1, mid)
        )
        sp, degp = _scatter([(h2, dst.reshape(eh // _CH, _CH))], n)
        sps.append(sp)
        degps.append(degp)
        ssums.append(ssum)
        ssqs.append(ssq)
    return _final(
        sps, degps, ssums, ssqs, bn_g.reshape(1, d), bn_b.reshape(1, d),
        Wr, br.reshape(1, d), x, e,
    )
